# baseline (speedup 1.0000x reference)
import functools

import jax
import jax.numpy as jnp
from jax import lax
from jax.experimental import pallas as pl
from jax.experimental.pallas import tpu as pltpu
from jax.experimental.pallas import tpu_sc as plsc

_NC = 2
_NS = 16
_NW = _NC * _NS


def _sc_gather(table, idx_flat):
    n_rows = idx_flat.shape[0]
    d = table.shape[1]
    per_w = n_rows // _NW
    ch = 128
    n_ch = per_w // ch
    assert n_ch % 2 == 0 and per_w % ch == 0

    mesh = plsc.VectorSubcoreMesh(core_axis_name="c", subcore_axis_name="s")

    @functools.partial(
        pl.kernel,
        out_type=jax.ShapeDtypeStruct((n_rows, d), jnp.float32),
        mesh=mesh,
        scratch_types=[
            pltpu.VMEM((per_w,), jnp.int32),
            pltpu.VMEM((ch, d), jnp.float32),
            pltpu.VMEM((ch, d), jnp.float32),
            pltpu.SemaphoreType.DMA,
            pltpu.SemaphoreType.DMA,
            pltpu.SemaphoreType.DMA,
            pltpu.SemaphoreType.DMA,
        ],
        compiler_params=pltpu.CompilerParams(use_tc_tiling_on_sc=False),
    )
    def gather_kernel(table_hbm, idx_hbm, out_hbm, idx_v, buf0, buf1,
                      gsem0, gsem1, osem0, osem1):
        wid = lax.axis_index("s") * _NC + lax.axis_index("c")
        base = wid * per_w
        pltpu.sync_copy(idx_hbm.at[pl.ds(base, per_w)], idx_v)

        def start_gather(i, buf, sem):
            pltpu.async_copy(
                table_hbm.at[idx_v.at[pl.ds(i * ch, ch)]], buf, sem)

        def wait_gather(buf, sem):
            pltpu.make_async_copy(
                table_hbm.at[idx_v.at[pl.ds(0, ch)]], buf, sem).wait()

        def start_out(i, buf, sem):
            pltpu.async_copy(buf, out_hbm.at[pl.ds(base + i * ch, ch)], sem)

        def wait_out(buf, sem):
            pltpu.make_async_copy(
                buf, out_hbm.at[pl.ds(base, ch)], sem).wait()

        start_gather(0, buf0, gsem0)

        @pl.loop(0, n_ch, step=2)
        def _(i):
            @pl.when(i > 0)
            def _():
                wait_out(buf1, osem1)
            wait_gather(buf0, gsem0)
            start_out(i, buf0, osem0)
            start_gather(i + 1, buf1, gsem1)
            wait_gather(buf1, gsem1)
            start_out(i + 1, buf1, osem1)

            @pl.when(i + 2 < n_ch)
            def _():
                wait_out(buf0, osem0)
                start_gather(i + 2, buf0, gsem0)

        wait_out(buf0, osem0)
        wait_out(buf1, osem1)

    return gather_kernel(table, idx_flat)


def _tc_dense(tok2d, vis2d, pos_tiled, w_t, gamma, beta, blk):
    n_rows, d = tok2d.shape
    vdim = vis2d.shape[1]
    grid = n_rows // blk

    def body(tok_ref, vis_ref, pos_ref, w_ref, g_ref, b_ref, out_ref):
        proj = jnp.dot(vis_ref[...], w_ref[...],
                       preferred_element_type=jnp.float32)
        emb = tok_ref[...] + pos_ref[...] + proj
        mean = jnp.mean(emb, axis=1, keepdims=True)
        cent = emb - mean
        var = jnp.mean(cent * cent, axis=1, keepdims=True)
        normed = cent * lax.rsqrt(var + 1e-6)
        out_ref[...] = normed * g_ref[...] + b_ref[...]

    return pl.pallas_call(
        body,
        grid=(grid,),
        in_specs=[
            pl.BlockSpec((blk, d), lambda i: (i, 0)),
            pl.BlockSpec((blk, vdim), lambda i: (i, 0)),
            pl.BlockSpec((blk, d), lambda i: (0, 0)),
            pl.BlockSpec((vdim, d), lambda i: (0, 0)),
            pl.BlockSpec((1, d), lambda i: (0, 0)),
            pl.BlockSpec((1, d), lambda i: (0, 0)),
        ],
        out_specs=pl.BlockSpec((blk, d), lambda i: (i, 0)),
        out_shape=jax.ShapeDtypeStruct((n_rows, d), jnp.float32),
    )(tok2d, vis2d, pos_tiled, w_t, gamma, beta)


def kernel(seq, visual_features, token_table, pos_table, W_visual,
           ln_gamma, ln_beta):
    b, t = seq.shape
    d = token_table.shape[1]
    n = b * t
    idx_flat = seq.reshape(n).astype(jnp.int32)

    tok2d = _sc_gather(token_table, idx_flat)

    vis2d = visual_features.reshape(n, -1)
    blk = 1600
    pos_tiled = jnp.tile(pos_table[:t], (blk // t, 1))
    out2d = _tc_dense(tok2d, vis2d, pos_tiled, W_visual.T,
                      ln_gamma.reshape(1, d), ln_beta.reshape(1, d), blk)
    return out2d.reshape(b, t, d)
